# baseline (speedup 1.0000x reference)
import functools

import jax
import jax.numpy as jnp
from jax import lax
from jax.experimental import pallas as pl
from jax.experimental.pallas import tpu as pltpu
from jax.experimental.pallas import tpu_sc as plsc

BATCH = 16384
NUM_PROGRAMS = 100000
DIM = 32
N_FEATS = 16
OUT_DIM = 4 * DIM + N_FEATS

ROWS_PER_SLAB = 4
SLAB_W = ROWS_PER_SLAB * DIM
NUM_SLABS = NUM_PROGRAMS // ROWS_PER_SLAB

NUM_CORES = 2
NUM_SUBCORES = 16
NUM_WORKERS = NUM_CORES * NUM_SUBCORES
BPW = BATCH // NUM_WORKERS
CHUNK = 128
NCHUNK = BPW // CHUNK
LANES = 16

_mesh = plsc.VectorSubcoreMesh(core_axis_name="c", subcore_axis_name="s")


@functools.partial(
    pl.kernel,
    mesh=_mesh,
    out_type=jax.ShapeDtypeStruct((OUT_DIM, BATCH), jnp.float32),
    scratch_types=[
        pltpu.VMEM((BPW,), jnp.int32),
        pltpu.VMEM((CHUNK,), jnp.int32),
        pltpu.VMEM((CHUNK,), jnp.int32),
        pltpu.VMEM((CHUNK, SLAB_W), jnp.float32),
        pltpu.VMEM((CHUNK, SLAB_W), jnp.float32),
        pltpu.VMEM((N_FEATS, BPW), jnp.float32),
        pltpu.VMEM((4 * DIM, BPW), jnp.float32),
        pltpu.SemaphoreType.DMA,
        pltpu.SemaphoreType.DMA,
        pltpu.SemaphoreType.DMA,
    ],
    compiler_params=pltpu.CompilerParams(needs_layout_passes=False),
)
def _matchup_sc(idx_hbm, feats_hbm, pw_hbm, tw_hbm, out_hbm,
                icol, sidA, sidB, slabA, slabB, fv, outv, gsemA, gsemB, fsem):
    wid = lax.axis_index("s") * NUM_CORES + lax.axis_index("c")
    base = wid * BPW

    cf = pltpu.async_copy(feats_hbm.at[:, pl.ds(base, BPW)], fv, fsem)

    def process_col(col, table_hbm):
        frow = col * DIM
        pltpu.sync_copy(idx_hbm.at[pl.ds(col * BATCH + base, BPW)], icol)

        def fill_sids(ch, sid_ref):
            def body(i):
                v = icol[pl.ds(ch * CHUNK + i * LANES, LANES)]
                sid_ref[pl.ds(i * LANES, LANES)] = jax.lax.shift_right_logical(
                    v, 2)
            for i in range(CHUNK // LANES):
                body(i)

        def start_gather(ch, sid_ref, slab_ref, sem):
            fill_sids(ch, sid_ref)
            return pltpu.async_copy(table_hbm.at[sid_ref], slab_ref, sem)

        def extract(ch, slab_ref):
            def group_body(i, _):
                b0 = ch * CHUNK + i * LANES
                v = icol[pl.ds(b0, LANES)]
                sub = jax.lax.bitwise_and(v, 3)
                rows = jax.lax.iota(jnp.int32, LANES) + i * LANES
                colbase = sub * DIM
                for f in range(DIM):
                    vals = plsc.load_gather(slab_ref, [rows, colbase + f])
                    outv[frow + f, pl.ds(b0, LANES)] = vals
                return ()
            jax.lax.fori_loop(0, CHUNK // LANES, group_body, ())

        h0 = start_gather(0, sidA, slabA, gsemA)
        h1 = start_gather(1, sidB, slabB, gsemB)
        h0.wait()
        extract(0, slabA)
        h2 = start_gather(2, sidA, slabA, gsemA)
        h1.wait()
        extract(1, slabB)
        h3 = start_gather(3, sidB, slabB, gsemB)
        h2.wait()
        extract(2, slabA)
        h3.wait()
        extract(3, slabB)

    process_col(0, pw_hbm)
    process_col(1, tw_hbm)
    process_col(2, pw_hbm)
    process_col(3, tw_hbm)

    pltpu.sync_copy(outv, out_hbm.at[pl.ds(0, 4 * DIM), pl.ds(base, BPW)])
    cf.wait()
    pltpu.sync_copy(fv, out_hbm.at[pl.ds(4 * DIM, N_FEATS), pl.ds(base, BPW)])


def kernel(x, program_weight, team_weight):
    idx_flat = x[:, :4].astype(jnp.int32).T.reshape(-1)
    feats_t = x[:, 4:].T
    pw_slabs = program_weight.reshape(NUM_SLABS, SLAB_W)
    tw_slabs = team_weight[:NUM_PROGRAMS].reshape(NUM_SLABS, SLAB_W)
    out_t = _matchup_sc(idx_flat, feats_t, pw_slabs, tw_slabs)
    return out_t.T
